# baseline (speedup 1.0000x reference)
import functools

import jax
import jax.numpy as jnp
from jax import lax
from jax.experimental import pallas as pl
from jax.experimental.pallas import tpu as pltpu
from jax.experimental.pallas import tpu_sc as plsc

_EMB = 64
_BATCH = 4096
_HIST = 50
_L = 16
_NLG = _EMB // _L
_CPG = 2
_GROWS = _CPG * _HIST


@functools.lru_cache(maxsize=None)
def _make_sc_kernel(NC, NS):
    NW = NC * NS
    BPW = _BATCH // NW
    NSUB = BPW // _CPG

    mesh = plsc.VectorSubcoreMesh(core_axis_name="c", subcore_axis_name="s")

    @functools.partial(
        pl.kernel,
        mesh=mesh,
        out_type=(
            jax.ShapeDtypeStruct((_BATCH, _EMB), jnp.float32),
            jax.ShapeDtypeStruct((_BATCH, _EMB), jnp.float32),
        ),
        compiler_params=pltpu.CompilerParams(use_tc_tiling_on_sc=False),
        scratch_types=[
            pltpu.VMEM((BPW,), jnp.int32),
            pltpu.VMEM((BPW, _EMB), jnp.float32),
            pltpu.VMEM((NSUB, _GROWS), jnp.int32),
            pltpu.VMEM((_GROWS, _EMB), jnp.float32),
            pltpu.VMEM((_GROWS, _EMB), jnp.float32),
            pltpu.VMEM((BPW, _EMB), jnp.float32),
            pltpu.SemaphoreType.DMA,
            pltpu.SemaphoreType.DMA,
            pltpu.SemaphoreType.DMA,
        ],
    )
    def k(tgt_hbm, ctx_hbm, emb_hbm, tgt_out, ctx_out,
          tidx_v, trows_v, cidx_v, buf0, buf1, ostage, tsem, sem0, sem1):
        cid = lax.axis_index("c")
        sid = lax.axis_index("s")
        wid = sid * NC + cid
        base = wid * BPW

        pltpu.sync_copy(tgt_hbm.at[wid], tidx_v)
        tgather = pltpu.make_async_copy(emb_hbm.at[tidx_v], trows_v, tsem)
        tgather.start()

        pltpu.sync_copy(ctx_hbm.at[wid], cidx_v)

        bufs = (buf0, buf1)
        sems = (sem0, sem1)

        def gather_chunk(si, ph):
            pltpu.make_async_copy(
                emb_hbm.at[cidx_v.at[si]], bufs[ph], sems[ph]).start()

        def wait_chunk(si, ph):
            pltpu.make_async_copy(
                emb_hbm.at[cidx_v.at[si]], bufs[ph], sems[ph]).wait()

        for ph in range(2):
            gather_chunk(ph, ph)

        def reduce_chunk(si, ph):
            buf = bufs[ph]
            for b in range(_CPG):
                def jbody(j, accs, b=b):
                    row = b * _HIST + j
                    return tuple(
                        accs[g] + buf[row, pl.ds(g * _L, _L)]
                        for g in range(_NLG))
                accs = lax.fori_loop(
                    0, _HIST, jbody,
                    tuple(jnp.zeros((_L,), jnp.float32) for _ in range(_NLG)))
                orow = si * _CPG + b
                for g in range(_NLG):
                    ostage[orow, pl.ds(g * _L, _L)] = accs[g]

        def sbody(si2, carry):
            for ph in range(2):
                si = si2 * 2 + ph
                wait_chunk(si, ph)
                reduce_chunk(si, ph)

                @pl.when(si + 2 < NSUB)
                def _():
                    gather_chunk(si + 2, ph)
            return carry

        lax.fori_loop(0, NSUB // 2, sbody, 0)

        tgather.wait()
        pltpu.sync_copy(trows_v, tgt_out.at[pl.ds(base, BPW)])
        pltpu.sync_copy(ostage, ctx_out.at[pl.ds(base, BPW)])

    return k


def kernel(target, context, emb):
    info = plsc.get_sparse_core_info()
    NC, NS = info.num_cores, info.num_subcores
    NW = NC * NS
    k = _make_sc_kernel(NC, NS)
    bpw = _BATCH // NW
    tgt_r = target.astype(jnp.int32).reshape(NW, bpw)
    ctx_r = context.astype(jnp.int32).reshape(NW, bpw // _CPG, _GROWS)
    return k(tgt_r, ctx_r, emb)


# speedup vs baseline: 1.0274x; 1.0274x over previous
import functools

import jax
import jax.numpy as jnp
from jax import lax
from jax.experimental import pallas as pl
from jax.experimental.pallas import tpu as pltpu
from jax.experimental.pallas import tpu_sc as plsc

_EMB = 64
_BATCH = 4096
_HIST = 50
_L = 16
_NLG = _EMB // _L
_CPG = 8
_GROWS = _CPG * _HIST


@functools.lru_cache(maxsize=None)
def _make_sc_kernel(NC, NS):
    NW = NC * NS
    BPW = _BATCH // NW
    NSUB = BPW // _CPG

    mesh = plsc.VectorSubcoreMesh(core_axis_name="c", subcore_axis_name="s")

    @functools.partial(
        pl.kernel,
        mesh=mesh,
        out_type=(
            jax.ShapeDtypeStruct((_BATCH, _EMB), jnp.float32),
            jax.ShapeDtypeStruct((_BATCH, _EMB), jnp.float32),
        ),
        compiler_params=pltpu.CompilerParams(use_tc_tiling_on_sc=False),
        scratch_types=[
            pltpu.VMEM((BPW,), jnp.int32),
            pltpu.VMEM((BPW, _EMB), jnp.float32),
            pltpu.VMEM((NSUB, _GROWS), jnp.int32),
            pltpu.VMEM((_GROWS, _EMB), jnp.float32),
            pltpu.VMEM((_GROWS, _EMB), jnp.float32),
            pltpu.VMEM((BPW, _EMB), jnp.float32),
            pltpu.SemaphoreType.DMA,
            pltpu.SemaphoreType.DMA,
            pltpu.SemaphoreType.DMA,
        ],
    )
    def k(tgt_hbm, ctx_hbm, emb_hbm, tgt_out, ctx_out,
          tidx_v, trows_v, cidx_v, buf0, buf1, ostage, tsem, sem0, sem1):
        cid = lax.axis_index("c")
        sid = lax.axis_index("s")
        wid = sid * NC + cid
        base = wid * BPW

        pltpu.sync_copy(tgt_hbm.at[wid], tidx_v)
        tgather = pltpu.make_async_copy(emb_hbm.at[tidx_v], trows_v, tsem)
        tgather.start()

        pltpu.sync_copy(ctx_hbm.at[wid], cidx_v)

        bufs = (buf0, buf1)
        sems = (sem0, sem1)

        def gather_chunk(si, ph):
            pltpu.make_async_copy(
                emb_hbm.at[cidx_v.at[si]], bufs[ph], sems[ph]).start()

        def wait_chunk(si, ph):
            pltpu.make_async_copy(
                emb_hbm.at[cidx_v.at[si]], bufs[ph], sems[ph]).wait()

        for ph in range(2):
            gather_chunk(ph, ph)

        def reduce_chunk(si, ph):
            buf = bufs[ph]
            for b in range(_CPG):
                def jbody(j, accs, b=b):
                    row = b * _HIST + j
                    return tuple(
                        accs[g] + buf[row, pl.ds(g * _L, _L)]
                        for g in range(_NLG))
                accs = lax.fori_loop(
                    0, _HIST, jbody,
                    tuple(jnp.zeros((_L,), jnp.float32) for _ in range(_NLG)))
                orow = si * _CPG + b
                for g in range(_NLG):
                    ostage[orow, pl.ds(g * _L, _L)] = accs[g]

        def sbody(si2, carry):
            for ph in range(2):
                si = si2 * 2 + ph
                wait_chunk(si, ph)
                reduce_chunk(si, ph)

                @pl.when(si + 2 < NSUB)
                def _():
                    gather_chunk(si + 2, ph)
            return carry

        lax.fori_loop(0, NSUB // 2, sbody, 0)

        tgather.wait()
        pltpu.sync_copy(trows_v, tgt_out.at[pl.ds(base, BPW)])
        pltpu.sync_copy(ostage, ctx_out.at[pl.ds(base, BPW)])

    return k


def kernel(target, context, emb):
    info = plsc.get_sparse_core_info()
    NC, NS = info.num_cores, info.num_subcores
    NW = NC * NS
    k = _make_sc_kernel(NC, NS)
    bpw = _BATCH // NW
    tgt_r = target.astype(jnp.int32).reshape(NW, bpw)
    ctx_r = context.astype(jnp.int32).reshape(NW, bpw // _CPG, _GROWS)
    return k(tgt_r, ctx_r, emb)


# speedup vs baseline: 1.0300x; 1.0025x over previous
import functools

import jax
import jax.numpy as jnp
from jax import lax
from jax.experimental import pallas as pl
from jax.experimental.pallas import tpu as pltpu
from jax.experimental.pallas import tpu_sc as plsc

_EMB = 64
_BATCH = 4096
_HIST = 50
_L = 16
_NLG = _EMB // _L
_CPG = 4
_GROWS = _CPG * _HIST
_NBUF = 4


@functools.lru_cache(maxsize=None)
def _make_sc_kernel(NC, NS):
    NW = NC * NS
    BPW = _BATCH // NW
    NSUB = BPW // _CPG

    mesh = plsc.VectorSubcoreMesh(core_axis_name="c", subcore_axis_name="s")

    @functools.partial(
        pl.kernel,
        mesh=mesh,
        out_type=(
            jax.ShapeDtypeStruct((_BATCH, _EMB), jnp.float32),
            jax.ShapeDtypeStruct((_BATCH, _EMB), jnp.float32),
        ),
        compiler_params=pltpu.CompilerParams(use_tc_tiling_on_sc=False),
        scratch_types=[
            pltpu.VMEM((BPW,), jnp.int32),
            pltpu.VMEM((BPW, _EMB), jnp.float32),
            pltpu.VMEM((NSUB, _GROWS), jnp.int32),
            pltpu.VMEM((BPW, _EMB), jnp.float32),
            pltpu.SemaphoreType.DMA,
        ]
        + [pltpu.VMEM((_GROWS, _EMB), jnp.float32) for _ in range(_NBUF)]
        + [pltpu.SemaphoreType.DMA for _ in range(_NBUF)],
    )
    def k(tgt_hbm, ctx_hbm, emb_hbm, tgt_out, ctx_out,
          tidx_v, trows_v, cidx_v, ostage, tsem, *ring):
        bufs = ring[:_NBUF]
        sems = ring[_NBUF:]
        cid = lax.axis_index("c")
        sid = lax.axis_index("s")
        wid = sid * NC + cid
        base = wid * BPW

        pltpu.sync_copy(tgt_hbm.at[wid], tidx_v)
        tgather = pltpu.make_async_copy(emb_hbm.at[tidx_v], trows_v, tsem)
        tgather.start()

        pltpu.sync_copy(ctx_hbm.at[wid], cidx_v)

        def gather_chunk(si, ph):
            pltpu.make_async_copy(
                emb_hbm.at[cidx_v.at[si]], bufs[ph], sems[ph]).start()

        def wait_chunk(ph):
            pltpu.make_async_copy(
                emb_hbm.at[cidx_v.at[0]], bufs[ph], sems[ph]).wait()

        for ph in range(_NBUF):
            gather_chunk(ph, ph)

        def reduce_chunk(si, ph):
            buf = bufs[ph]
            for b in range(_CPG):
                def jbody(j, accs, b=b):
                    row = b * _HIST + j
                    return tuple(
                        accs[g] + buf[row, pl.ds(g * _L, _L)]
                        for g in range(_NLG))
                accs = lax.fori_loop(
                    0, _HIST, jbody,
                    tuple(jnp.zeros((_L,), jnp.float32) for _ in range(_NLG)))
                orow = si * _CPG + b
                for g in range(_NLG):
                    ostage[orow, pl.ds(g * _L, _L)] = accs[g]

        def sbody(si2, carry):
            for ph in range(_NBUF):
                si = si2 * _NBUF + ph
                wait_chunk(ph)
                reduce_chunk(si, ph)

                @pl.when(si + _NBUF < NSUB)
                def _():
                    gather_chunk(si + _NBUF, ph)
            return carry

        lax.fori_loop(0, NSUB // _NBUF, sbody, 0)

        tgather.wait()
        pltpu.sync_copy(trows_v, tgt_out.at[pl.ds(base, BPW)])
        pltpu.sync_copy(ostage, ctx_out.at[pl.ds(base, BPW)])

    return k


def kernel(target, context, emb):
    info = plsc.get_sparse_core_info()
    NC, NS = info.num_cores, info.num_subcores
    NW = NC * NS
    k = _make_sc_kernel(NC, NS)
    bpw = _BATCH // NW
    tgt_r = target.astype(jnp.int32).reshape(NW, bpw)
    ctx_r = context.astype(jnp.int32).reshape(NW, bpw // _CPG, _GROWS)
    return k(tgt_r, ctx_r, emb)
